# baseline (speedup 1.0000x reference)
import numpy as np
import jax
import jax.numpy as jnp
from jax.experimental import pallas as pl
from jax.experimental.pallas import tpu as pltpu

_HEIGHT = 0.5
_TOP_UNIT = None


def _top_unit():
    global _TOP_UNIT
    if _TOP_UNIT is None:
        with jax.ensure_compile_time_eval():
            _TOP_UNIT = float(jax.random.uniform(jax.random.key(1), ()))
    return _TOP_UNIT


def kernel(img_batch, labels, cu_seqlens):
    N, C, H, W = img_batch.shape
    total = labels.shape[0]
    crop_h = int(H * _HEIGHT)
    top = np.float32(_top_unit()) * np.float32(1.0 - _HEIGHT)
    top_px = np.float32(top * np.float32(H))
    bottom_px = np.float32(top_px + np.float32(H * _HEIGHT))
    top_idx = int(np.floor(top_px))

    R = total // 128
    lab_t = labels.T.reshape(5, R, 128)

    def body(cu_ref, img_ref, lab_ref, img_out_ref, lab_out_ref, cnt_ref, sem):
        cp = pltpu.make_async_copy(
            img_ref.at[:, :, pl.ds(top_idx, crop_h), :], img_out_ref, sem)
        cp.start()
        cls = lab_ref[0]
        cx = lab_ref[1]
        cy = lab_ref[2]
        w = lab_ref[3]
        h = lab_ref[4]
        tpx = jnp.float32(top_px)
        bpx = jnp.float32(bottom_px)
        inside = (cy > tpx) & (cy < bpx)
        half = h * jnp.float32(0.5)
        y1c = jnp.maximum(cy - half, tpx)
        y2c = jnp.minimum(cy + half, bpx)
        ncy = (y1c + y2c) * jnp.float32(0.5)
        nh = y2c - y1c
        insf = inside.astype(jnp.float32)
        lab_out_ref[0] = cls * insf
        lab_out_ref[1] = cx * insf
        lab_out_ref[2] = ncy * insf
        lab_out_ref[3] = w * insf
        lab_out_ref[4] = nh * insf
        pos = (jax.lax.broadcasted_iota(jnp.int32, (R, 128), 0) * 128
               + jax.lax.broadcasted_iota(jnp.int32, (R, 128), 1))
        insi = inside.astype(jnp.int32)
        for i in range(N):
            lo = cu_ref[i]
            hi = cu_ref[i + 1]
            m = (pos >= lo) & (pos < hi)
            cnt_ref[0, i] = jnp.sum(jnp.where(m, insi, 0))
        cp.wait()

    img_out, lab_out, counts = pl.pallas_call(
        body,
        in_specs=[
            pl.BlockSpec(memory_space=pltpu.SMEM),
            pl.BlockSpec(memory_space=pltpu.MemorySpace.HBM),
            pl.BlockSpec(memory_space=pltpu.VMEM),
        ],
        out_specs=[
            pl.BlockSpec(memory_space=pltpu.MemorySpace.HBM),
            pl.BlockSpec(memory_space=pltpu.VMEM),
            pl.BlockSpec(memory_space=pltpu.SMEM),
        ],
        out_shape=[
            jax.ShapeDtypeStruct((N, C, crop_h, W), img_batch.dtype),
            jax.ShapeDtypeStruct((5, R, 128), labels.dtype),
            jax.ShapeDtypeStruct((1, N), jnp.int32),
        ],
        scratch_shapes=[pltpu.SemaphoreType.DMA],
    )(cu_seqlens, img_batch, lab_t)

    new_labels = lab_out.reshape(5, total).T
    counts = counts.reshape(N)
    return img_out, new_labels, counts
